# baseline (speedup 1.0000x reference)
import functools

import jax
import jax.numpy as jnp
from jax.experimental import pallas as pl
from jax.experimental.pallas import tpu as pltpu

_NEG_INF = -1e30
_KERNEL_WINS = (3, 4, 5)
_DIM_CHANNEL = 256
_PAD_ID = 0
_NUM_CLASS = 20


def _round_up(x, m):
    return ((x + m - 1) // m) * m


def _fused_kernel(xr_ref, xcol_ref, mpad_ref, tbl_ref, wtap_ref,
                  w1_ref, b1_ref, w2_ref, b2_ref, out_ref, gbuf,
                  *, kernel_wins, dim_channel, bt, l_seq, l_ext):
    Bt, L, L_ext = bt, l_seq, l_ext
    E = tbl_ref.shape[2]
    M = Bt * L_ext
    C = dim_channel

    UN = next(u for u in (40, 32, 16, 8, 4, 2, 1) if M % u == 0)

    def gather_chunk(c, _):
        for i in range(UN):
            k = c * UN + i
            gbuf[k] = tbl_ref[xr_ref[0, 0, k]]
        return 0

    jax.lax.fori_loop(0, M // UN, gather_chunk, 0)

    odd = (xcol_ref[...] & 1) == 1
    g = gbuf[...].reshape(M, E)
    sel = jnp.where(odd, pltpu.roll(g, E // 2, axis=1), g)
    lo = pltpu.bitcast(sel << 16, jnp.float32).astype(jnp.bfloat16)
    hi = pltpu.bitcast(sel & jnp.int32(-65536),
                       jnp.float32).astype(jnp.bfloat16)
    emb = jnp.concatenate([lo[:, :E // 2], hi[:, :E // 2]], axis=1)
    emb = emb.reshape(Bt, L_ext, E)

    n_taps = L_ext - L + 1
    unf = jnp.concatenate([emb[:, k:k + L, :] for k in range(n_taps)],
                          axis=-1).reshape(Bt * L, n_taps * E)

    mpad = mpad_ref[...]
    pooled = []
    for i, win in enumerate(kernel_wins):
        depth = win * E
        a = jnp.dot(unf[:, :depth], wtap_ref[:depth, i * C:(i + 1) * C],
                    preferred_element_type=jnp.float32)
        a = a.reshape(Bt, L, C)
        if win > 1:
            m = jnp.concatenate(
                [mpad[:, win - 1:],
                 jnp.full((Bt, win - 1), _NEG_INF, jnp.float32)], axis=1)
        else:
            m = mpad
        pooled.append(jnp.max(a + m[:, :, None], axis=1))
    pooled = jnp.concatenate(pooled, axis=-1)

    h = jnp.dot(pooled, w1_ref[...], preferred_element_type=jnp.float32)
    h = jnp.maximum(h + b1_ref[...], 0.0)
    out_ref[...] = jnp.dot(h, w2_ref[...],
                           preferred_element_type=jnp.float32) + b2_ref[...]


@jax.jit
def _forward(embed, wtap, w1, b1, w2, b2, x_ids):
    B, L = x_ids.shape
    E = embed.shape[1]
    KP, CP = wtap.shape
    n_taps = KP // E
    FP = w1.shape[1]
    NCP = w2.shape[1]
    L_ext = L + n_taps - 1

    Bt = 8
    B_pad = _round_up(B, Bt)
    NB = B_pad // Bt
    M = Bt * L_ext
    grid = (NB,)

    x_ext = jnp.pad(x_ids, ((0, B_pad - B), (0, n_taps - 1)),
                    constant_values=_PAD_ID)
    xf = x_ext.reshape(-1)
    xrow = (xf >> 1).reshape(NB, 1, M)
    xcol = xf.reshape(B_pad * L_ext, 1)
    V = embed.shape[0]
    tbl3 = jax.lax.bitcast_convert_type(
        embed.reshape(V // 2, E, 2), jnp.int32).reshape(V // 2, 1, E)
    perm = jnp.concatenate([jnp.arange(0, E, 2, dtype=jnp.int32),
                            jnp.arange(1, E, 2, dtype=jnp.int32)])
    wtap_p = wtap.reshape(n_taps, E, CP)[:, perm, :].reshape(KP, CP)
    mpad = jnp.where(x_ext[:, :L] == _PAD_ID,
                     jnp.float32(-100.0), jnp.float32(0.0))

    kern = functools.partial(_fused_kernel, kernel_wins=_KERNEL_WINS,
                             dim_channel=_DIM_CHANNEL, bt=Bt, l_seq=L,
                             l_ext=L_ext)
    out = pl.pallas_call(
        kern,
        out_shape=jax.ShapeDtypeStruct((B_pad, NCP), jnp.float32),
        grid=grid,
        in_specs=[
            pl.BlockSpec((1, 1, M), lambda b: (b, 0, 0),
                         memory_space=pltpu.SMEM),
            pl.BlockSpec((M, 1), lambda b: (b, 0)),
            pl.BlockSpec((Bt, L), lambda b: (b, 0)),
            pl.BlockSpec((V // 2, 1, E), lambda b: (0, 0, 0)),
            pl.BlockSpec((KP, CP), lambda b: (0, 0)),
            pl.BlockSpec((CP, FP), lambda b: (0, 0)),
            pl.BlockSpec((1, FP), lambda b: (0, 0)),
            pl.BlockSpec((FP, NCP), lambda b: (0, 0)),
            pl.BlockSpec((1, NCP), lambda b: (0, 0)),
        ],
        out_specs=pl.BlockSpec((Bt, NCP), lambda b: (b, 0)),
        scratch_shapes=[pltpu.VMEM((M, 1, E), jnp.int32)],
        compiler_params=pltpu.CompilerParams(
            dimension_semantics=("parallel",),
            vmem_limit_bytes=60 * 1024 * 1024),
    )(xrow, xcol, mpad, tbl3, wtap_p, w1, b1, w2, b2)

    return out[:B, :_NUM_CLASS]


def kernel(embed, wtap, w1, b1, w2, b2, x_ids):
    return _forward(embed, wtap, w1, b1, w2, b2, x_ids)


# speedup vs baseline: 11.2055x; 11.2055x over previous
import functools

import jax
import jax.numpy as jnp
from jax.experimental import pallas as pl
from jax.experimental.pallas import tpu as pltpu

_NEG_INF = -1e30
_KERNEL_WINS = (3, 4, 5)
_DIM_CHANNEL = 256
_PAD_ID = 0
_NUM_CLASS = 20


def _round_up(x, m):
    return ((x + m - 1) // m) * m


def _fused_kernel(xr_ref, xcol_ref, mpad_ref, tbl_ref, wtap_ref,
                  w1_ref, b1_ref, w2_ref, b2_ref, out_ref, gbuf, ebuf,
                  *, kernel_wins, dim_channel, bt, l_seq, l_ext):
    Bt, L, L_ext = bt, l_seq, l_ext
    E = tbl_ref.shape[2]
    M = Bt * L_ext
    C = dim_channel

    UN = next(u for u in (40, 32, 16, 8, 4, 2, 1) if M % u == 0)

    def gather_chunk(c, _):
        for i in range(UN):
            k = c * UN + i
            gbuf[k] = tbl_ref[xr_ref[0, 0, k]]
        return 0

    jax.lax.fori_loop(0, M // UN, gather_chunk, 0)

    G = next(g for g in (40, 32, 16, 8, 4, 2, 1) if M % g == 0)
    rows = xcol_ref[...] & 15
    iota_l = jax.lax.broadcasted_iota(jnp.int32, (G, 16 * G), 1)
    base = 16 * jax.lax.broadcasted_iota(jnp.int32, (G, 1), 0)
    for g in range(M // G):
        tgt = base + rows[g * G:(g + 1) * G]
        sel = (iota_l == tgt).astype(jnp.bfloat16)
        st = gbuf[g * G:(g + 1) * G].reshape(16 * G, E)
        e_g = jnp.dot(sel, st, preferred_element_type=jnp.float32)
        ebuf[g * G:(g + 1) * G, :] = e_g.astype(jnp.bfloat16)

    emb = ebuf[...].reshape(Bt, L_ext, E)

    n_taps = L_ext - L + 1
    unf = jnp.concatenate([emb[:, k:k + L, :] for k in range(n_taps)],
                          axis=-1).reshape(Bt * L, n_taps * E)

    mpad = mpad_ref[...]
    pooled = []
    for i, win in enumerate(kernel_wins):
        depth = win * E
        a = jnp.dot(unf[:, :depth], wtap_ref[:depth, i * C:(i + 1) * C],
                    preferred_element_type=jnp.float32)
        a = a.reshape(Bt, L, C)
        if win > 1:
            m = jnp.concatenate(
                [mpad[:, win - 1:],
                 jnp.full((Bt, win - 1), _NEG_INF, jnp.float32)], axis=1)
        else:
            m = mpad
        pooled.append(jnp.max(a + m[:, :, None], axis=1))
    pooled = jnp.concatenate(pooled, axis=-1)

    h = jnp.dot(pooled, w1_ref[...], preferred_element_type=jnp.float32)
    h = jnp.maximum(h + b1_ref[...], 0.0)
    out_ref[...] = jnp.dot(h, w2_ref[...],
                           preferred_element_type=jnp.float32) + b2_ref[...]


@jax.jit
def _forward(embed, wtap, w1, b1, w2, b2, x_ids):
    B, L = x_ids.shape
    E = embed.shape[1]
    KP, CP = wtap.shape
    n_taps = KP // E
    FP = w1.shape[1]
    NCP = w2.shape[1]
    L_ext = L + n_taps - 1

    Bt = 8
    B_pad = _round_up(B, Bt)
    NB = B_pad // Bt
    M = Bt * L_ext
    grid = (NB,)

    x_ext = jnp.pad(x_ids, ((0, B_pad - B), (0, n_taps - 1)),
                    constant_values=_PAD_ID)
    xf = x_ext.reshape(-1)
    xrow = (xf >> 4).reshape(NB, 1, M)
    xcol = xf.reshape(B_pad * L_ext, 1)
    V = embed.shape[0]
    tbl3 = embed.reshape(V // 16, 16, E)
    mpad = jnp.where(x_ext[:, :L] == _PAD_ID,
                     jnp.float32(-100.0), jnp.float32(0.0))

    kern = functools.partial(_fused_kernel, kernel_wins=_KERNEL_WINS,
                             dim_channel=_DIM_CHANNEL, bt=Bt, l_seq=L,
                             l_ext=L_ext)
    out = pl.pallas_call(
        kern,
        out_shape=jax.ShapeDtypeStruct((B_pad, NCP), jnp.float32),
        grid=grid,
        in_specs=[
            pl.BlockSpec((1, 1, M), lambda b: (b, 0, 0),
                         memory_space=pltpu.SMEM),
            pl.BlockSpec((M, 1), lambda b: (b, 0)),
            pl.BlockSpec((Bt, L), lambda b: (b, 0)),
            pl.BlockSpec((V // 16, 16, E), lambda b: (0, 0, 0)),
            pl.BlockSpec((KP, CP), lambda b: (0, 0)),
            pl.BlockSpec((CP, FP), lambda b: (0, 0)),
            pl.BlockSpec((1, FP), lambda b: (0, 0)),
            pl.BlockSpec((FP, NCP), lambda b: (0, 0)),
            pl.BlockSpec((1, NCP), lambda b: (0, 0)),
        ],
        out_specs=pl.BlockSpec((Bt, NCP), lambda b: (b, 0)),
        scratch_shapes=[pltpu.VMEM((M, 16, E), jnp.bfloat16),
                        pltpu.VMEM((M, E), jnp.bfloat16)],
        compiler_params=pltpu.CompilerParams(
            dimension_semantics=("parallel",),
            vmem_limit_bytes=60 * 1024 * 1024),
    )(xrow, xcol, mpad, tbl3, wtap, w1, b1, w2, b2)

    return out[:B, :_NUM_CLASS]


def kernel(embed, wtap, w1, b1, w2, b2, x_ids):
    return _forward(embed, wtap, w1, b1, w2, b2, x_ids)
